# baseline (speedup 1.0000x reference)
import jax
import jax.numpy as jnp
from jax import lax
from jax.experimental import pallas as pl
from jax.experimental.pallas import tpu as pltpu
from jax.experimental.pallas import tpu_sc as plsc

N = 6_400_000
K_KEEP = N // 2
NW = 32
PER_W = N // NW
BLK = 8_000
NBLK = PER_W // BLK
VPB = BLK // 16
UNROLL = 10

NB = 4096
FNB = float(NB)
INV1 = float(2.0 ** -12)
INV2 = float(2.0 ** -24)
INV3 = float(2.0 ** -36)

_MESH = plsc.VectorSubcoreMesh(core_axis_name="c", subcore_axis_name="s")
_SC_PARAMS = pltpu.CompilerParams(needs_layout_passes=False)


def _wid():
    return lax.axis_index("s") * 2 + lax.axis_index("c")


def _lane_reduce(hist_v, red_v, lane):
    def group_body(g, carry):
        acc = jnp.zeros((16,), jnp.int32)
        for j in range(16):
            s = jnp.sum(hist_v[pl.ds((g * 16 + j) * 16, 16)])
            acc = jnp.where(lane == j, jnp.full((16,), s, jnp.int32), acc)
        red_v[pl.ds(g * 16, 16)] = acc
        return carry

    lax.fori_loop(0, NB // 16, group_body, 0)


def _digit0(p):
    t0 = p * FNB
    d0 = jnp.minimum(t0.astype(jnp.int32), NB - 1)
    return t0, d0


def _digit1(t0, d0):
    r1 = (t0 - d0.astype(jnp.float32)) * FNB
    d1 = jnp.minimum(r1.astype(jnp.int32), NB - 1)
    return r1, d1


def _digit2(r1, d1):
    r2 = (r1 - d1.astype(jnp.float32)) * FNB
    d2 = jnp.minimum(r2.astype(jnp.int32), NB - 1)
    return d2


@pl.kernel(
    out_type=[
        jax.ShapeDtypeStruct((N,), jnp.float32),
        jax.ShapeDtypeStruct((NW, NB), jnp.int32),
    ],
    mesh=_MESH,
    compiler_params=_SC_PARAMS,
    scratch_types=[
        pltpu.VMEM((BLK,), jnp.int32),
        pltpu.VMEM((BLK,), jnp.int32),
        pltpu.VMEM((BLK,), jnp.float32),
        pltpu.VMEM((BLK,), jnp.float32),
        pltpu.VMEM((NB * 16,), jnp.int32),
        pltpu.VMEM((NB,), jnp.int32),
        pltpu.SemaphoreType.DMA,
        pltpu.SemaphoreType.DMA,
        pltpu.SemaphoreType.DMA,
        pltpu.SemaphoreType.DMA,
        pltpu.SemaphoreType.DMA,
        pltpu.SemaphoreType.DMA,
    ],
)
def _k1_gather_hist(table, ids, zeros, probs_out, hist_out,
                    idx0, idx1, val0, val1, hist_v, red_v,
                    sg0, sg1, sw0, sw1, si0, si1):
    wid = _wid()
    base = wid * PER_W
    lane = lax.iota(jnp.int32, 16)
    ones = jnp.ones((16,), jnp.int32)
    pltpu.sync_copy(zeros, hist_v)

    idx = (idx0, idx1)
    val = (val0, val1)
    sg = (sg0, sg1)
    sw = (sw0, sw1)
    si = (si0, si1)
    h_g = [None, None]
    h_w = [None, None]
    h_i = [None, None]

    pltpu.sync_copy(ids.at[pl.ds(base, BLK)], idx0)
    h_g[0] = pltpu.async_copy(table.at[idx0], val0, sg[0])
    pltpu.sync_copy(ids.at[pl.ds(base + BLK, BLK)], idx1)

    for b in range(NBLK):
        par = b & 1
        oth = 1 - par
        off = base + b * BLK
        if b + 1 < NBLK:
            if h_w[oth] is not None:
                h_w[oth].wait()
            h_g[oth] = pltpu.async_copy(table.at[idx[oth]], val[oth], sg[oth])
        h_g[par].wait()
        if b + 2 < NBLK:
            h_i[par] = pltpu.async_copy(
                ids.at[pl.ds(base + (b + 2) * BLK, BLK)], idx[par], si[par])
        valc = val[par]

        def vec_body(j, c2):
            x = valc[pl.ds(j * 16, 16)]
            p = 1.0 / (1.0 + jnp.exp(-x))
            valc[pl.ds(j * 16, 16)] = p
            _, d0 = _digit0(p)
            plsc.addupdate_scatter(hist_v, [d0 * 16 + lane], ones)
            return c2

        lax.fori_loop(0, VPB, vec_body, 0, unroll=UNROLL)
        h_w[par] = pltpu.async_copy(valc, probs_out.at[pl.ds(off, BLK)],
                                    sw[par])
        if b + 2 < NBLK:
            h_i[par].wait()

    h_w[0].wait()
    h_w[1].wait()
    _lane_reduce(hist_v, red_v, lane)
    pltpu.sync_copy(red_v, hist_out.at[wid])


def _make_hist_level(level):
    @pl.kernel(
        out_type=jax.ShapeDtypeStruct((NW, NB), jnp.int32),
        mesh=_MESH,
        compiler_params=_SC_PARAMS,
        scratch_types=[
            pltpu.VMEM((BLK,), jnp.float32),
            pltpu.VMEM((BLK,), jnp.float32),
            pltpu.VMEM((NB * 16,), jnp.int32),
            pltpu.VMEM((NB,), jnp.int32),
            pltpu.VMEM((2, 16), jnp.int32),
            pltpu.VMEM((2, 16), jnp.int32),
            pltpu.SemaphoreType.DMA,
            pltpu.SemaphoreType.DMA,
        ],
    )
    def _hist_level(probs, zeros, params1, params2, hist_out,
                    val0, val1, hist_v, red_v, par1_v, par2_v, s0, s1):
        wid = _wid()
        base = wid * PER_W
        lane = lax.iota(jnp.int32, 16)
        ones = jnp.ones((16,), jnp.int32)
        pltpu.sync_copy(zeros, hist_v)
        pltpu.sync_copy(params1, par1_v)
        pltpu.sync_copy(params2, par2_v)
        d0_want = par1_v[0, :]
        d1_want = par2_v[0, :]

        val = (val0, val1)
        sem = (s0, s1)
        h_r = [None, None]
        h_r[0] = pltpu.async_copy(probs.at[pl.ds(base, BLK)], val0, sem[0])

        for b in range(NBLK):
            par = b & 1
            oth = 1 - par
            if b + 1 < NBLK:
                h_r[oth] = pltpu.async_copy(
                    probs.at[pl.ds(base + (b + 1) * BLK, BLK)], val[oth],
                    sem[oth])
            h_r[par].wait()
            valc = val[par]

            def vec_body(j, c2):
                p = valc[pl.ds(j * 16, 16)]
                t0, d0 = _digit0(p)
                r1, d1 = _digit1(t0, d0)
                if level == 1:
                    m = d0 == d0_want
                    bins = d1
                else:
                    m = jnp.logical_and(d0 == d0_want, d1 == d1_want)
                    bins = _digit2(r1, d1)
                plsc.addupdate_scatter(hist_v, [bins * 16 + lane], ones,
                                       mask=m)
                return c2

            lax.fori_loop(0, VPB, vec_body, 0, unroll=UNROLL)

        _lane_reduce(hist_v, red_v, lane)
        pltpu.sync_copy(red_v, hist_out.at[wid])

    return _hist_level


_h1_hist = _make_hist_level(1)
_h2_hist = _make_hist_level(2)


@pl.kernel(
    out_type=[
        jax.ShapeDtypeStruct((N,), jnp.float32),
        jax.ShapeDtypeStruct((N,), jnp.float32),
    ],
    mesh=_MESH,
    compiler_params=_SC_PARAMS,
    scratch_types=[
        pltpu.VMEM((BLK,), jnp.float32),
        pltpu.VMEM((BLK,), jnp.float32),
        pltpu.VMEM((BLK,), jnp.float32),
        pltpu.VMEM((BLK,), jnp.float32),
        pltpu.VMEM((BLK,), jnp.float32),
        pltpu.VMEM((BLK,), jnp.float32),
        pltpu.VMEM((1, 16), jnp.float32),
        pltpu.VMEM((NW, 16), jnp.int32),
        pltpu.SemaphoreType.DMA,
        pltpu.SemaphoreType.DMA,
        pltpu.SemaphoreType.DMA,
        pltpu.SemaphoreType.DMA,
        pltpu.SemaphoreType.DMA,
        pltpu.SemaphoreType.DMA,
    ],
)
def _k4_mask(probs, tpar, bpar, hard_out, soft_out,
             val0, val1, hard0, hard1, soft0, soft1, tpar_v, bpar_v,
             sr0, sr1, sh0, sh1, ss0, ss1):
    wid = _wid()
    base = wid * PER_W
    lane = lax.iota(jnp.int32, 16)
    pltpu.sync_copy(tpar, tpar_v)
    pltpu.sync_copy(bpar, bpar_v)
    t_v = tpar_v[0, :]
    b_v = jnp.zeros((16,), jnp.int32)
    for i in range(NW):
        b_v = jnp.where(wid == i, bpar_v[i, :], b_v)
    b_s = jnp.sum(jnp.where(lane == 0, b_v, 0))

    val = (val0, val1)
    hbuf = (hard0, hard1)
    sbuf = (soft0, soft1)
    sr = (sr0, sr1)
    sh = (sh0, sh1)
    ss = (ss0, ss1)
    h_r = [None, None]
    h_h = [None, None]
    h_s = [None, None]
    h_r[0] = pltpu.async_copy(probs.at[pl.ds(base, BLK)], val0, sr[0])

    seen = jnp.int32(0)
    for b in range(NBLK):
        par = b & 1
        oth = 1 - par
        off = base + b * BLK
        if b + 1 < NBLK:
            h_r[oth] = pltpu.async_copy(
                probs.at[pl.ds(base + (b + 1) * BLK, BLK)], val[oth], sr[oth])
        h_r[par].wait()
        if h_h[par] is not None:
            h_h[par].wait()
            h_s[par].wait()
        valc = val[par]
        hardc = hbuf[par]
        softc = sbuf[par]
        seen_v = jnp.full((16,), seen, jnp.int32)
        assume_v = seen_v < b_v

        def vec_body(j, acc):
            p = valc[pl.ds(j * 16, 16)]
            gt = p > t_v
            tie = p == t_v
            keep = jnp.logical_or(gt, jnp.logical_and(tie, assume_v))
            hard = jnp.where(keep, jnp.float32(1.0), jnp.float32(0.0))
            hardc[pl.ds(j * 16, 16)] = hard
            softc[pl.ds(j * 16, 16)] = (hard - p) + p
            return acc + plsc.all_reduce_population_count(tie)

        acc = lax.fori_loop(0, VPB, vec_body,
                            jnp.zeros((16,), jnp.int32), unroll=UNROLL)
        cnt = jnp.sum(jnp.where(lane == 0, acc, 0))
        redo = jnp.logical_and(seen < b_s, seen + cnt > b_s)

        def fix_block(seen_in):
            def fix_body(j, s2):
                p = valc[pl.ds(j * 16, 16)]
                gt = p > t_v
                tie = p == t_v
                t_i = tie.astype(jnp.int32)
                incl = plsc.cumsum(t_i)
                rank = incl - t_i + jnp.full((16,), s2, jnp.int32)
                keep = jnp.logical_or(gt, jnp.logical_and(tie, rank < b_v))
                hard = jnp.where(keep, jnp.float32(1.0), jnp.float32(0.0))
                hardc[pl.ds(j * 16, 16)] = hard
                softc[pl.ds(j * 16, 16)] = (hard - p) + p
                return s2 + jnp.sum(t_i)

            lax.fori_loop(0, VPB, fix_body, seen_in)
            return 0

        lax.cond(redo, lambda: fix_block(seen), lambda: 0)
        h_h[par] = pltpu.async_copy(hardc, hard_out.at[pl.ds(off, BLK)],
                                    sh[par])
        h_s[par] = pltpu.async_copy(softc, soft_out.at[pl.ds(off, BLK)],
                                    ss[par])
        seen = seen + cnt

    for par in (0, 1):
        if h_h[par] is not None:
            h_h[par].wait()
            h_s[par].wait()


def _suffix_sum(x):
    sh = 1
    while sh < NB:
        pad = jnp.zeros((1, sh), jnp.int32)
        x = x + jnp.concatenate([x[:, sh:], pad], axis=1)
        sh *= 2
    return x


def _select_tc_body(hist_ref, params_ref, out_ref):
    k_rem = params_ref[1, 0]
    h = jnp.sum(hist_ref[...], axis=0, keepdims=True)
    s = _suffix_sum(h)
    iota = lax.broadcasted_iota(jnp.int32, (1, NB), 1)
    mask = s >= k_rem
    hi = jnp.max(jnp.where(mask, iota, -1))
    s_h = jnp.min(jnp.where(mask, s, jnp.int32(2**31 - 1)))
    h_h = jnp.sum(jnp.where(iota == hi, h, 0))
    new_k = k_rem - (s_h - h_h)
    out_ref[...] = jnp.concatenate(
        [jnp.full((1, 16), hi, jnp.int32),
         jnp.full((1, 16), new_k, jnp.int32)], axis=0)


_t_select = pl.pallas_call(
    _select_tc_body, out_shape=jax.ShapeDtypeStruct((2, 16), jnp.int32))


def _t3_budget_body(hist_ref, p1_ref, p2_ref, tout_ref, bout_ref):
    d0 = p1_ref[0, 0]
    d1 = p2_ref[0, 0]
    k_rem = p2_ref[1, 0]
    hh = hist_ref[...]
    h = jnp.sum(hh, axis=0, keepdims=True)
    s = _suffix_sum(h)
    iota = lax.broadcasted_iota(jnp.int32, (1, NB), 1)
    mask = s >= k_rem
    d2 = jnp.max(jnp.where(mask, iota, -1))
    s_h = jnp.min(jnp.where(mask, s, jnp.int32(2**31 - 1)))
    h_h = jnp.sum(jnp.where(iota == d2, h, 0))
    r = k_rem - (s_h - h_h)
    t_val = (d0.astype(jnp.float32) * INV1
             + d1.astype(jnp.float32) * INV2
             + d2.astype(jnp.float32) * INV3)
    c = jnp.sum(jnp.where(iota == d2, hh, 0), axis=1, keepdims=True)
    x = c
    sh = 1
    while sh < NW:
        pad = jnp.zeros((sh, 1), jnp.int32)
        x = x + jnp.concatenate([pad, x[:-sh, :]], axis=0)
        sh *= 2
    p_excl = x - c
    budget = jnp.clip(r - p_excl, 0, c)
    tout_ref[...] = jnp.full((1, 16), t_val, jnp.float32)
    bout_ref[...] = jnp.broadcast_to(budget, (NW, 16))


_t3_budget = pl.pallas_call(
    _t3_budget_body,
    out_shape=[jax.ShapeDtypeStruct((1, 16), jnp.float32),
               jax.ShapeDtypeStruct((NW, 16), jnp.int32)])


def kernel(logits_weight, edge_ids):
    table = logits_weight.reshape(-1)
    zeros_hist = jnp.zeros((NB * 16,), jnp.int32)
    probs, hist0 = _k1_gather_hist(table, edge_ids, zeros_hist)
    params0 = jnp.concatenate(
        [jnp.zeros((1, 16), jnp.int32),
         jnp.full((1, 16), K_KEEP, jnp.int32)], axis=0)
    p1 = _t_select(hist0, params0)
    hist1 = _h1_hist(probs, zeros_hist, p1, p1)
    p2 = _t_select(hist1, p1)
    hist2 = _h2_hist(probs, zeros_hist, p1, p2)
    t_f32, budgets = _t3_budget(hist2, p1, p2)
    hard, soft = _k4_mask(probs, t_f32, budgets)
    return probs, soft, hard
